# baseline (speedup 1.0000x reference)
import jax
import jax.numpy as jnp
from jax.experimental import pallas as pl
from jax.experimental.pallas import tpu as pltpu

H = 16777216
NG = 64
COLS = 1024
ROWS = H // COLS
BLK_ROWS = 256
NBLK = ROWS // BLK_ROWS
STRIDE = 266305
OFF_IN_BLK = STRIDE - BLK_ROWS * COLS


def _body(x_ref, gw_ref, sc_ref, lc_ref, out_ref, scr_ref):
    g = pl.program_id(0)
    b = jax.lax.rem(g + 1, NBLK)
    off = OFF_IN_BLK * b
    r = off // COLS
    c = jax.lax.rem(off, COLS)

    xblk = x_ref[...]
    out_ref[...] = xblk

    lane = jax.lax.broadcasted_iota(jnp.int32, (1, COLS), 1)
    l64 = jax.lax.broadcasted_iota(jnp.int32, (1, NG), 1)

    xrow = x_ref[pl.ds(r, 1), :]
    xval = jnp.sum(jnp.where(lane == c, xrow, 0.0))
    gwb = jnp.sum(jnp.where(l64 == b, gw_ref[...], 0.0))
    scb = jnp.sum(jnp.where(l64 == b, sc_ref[...], 0.0))
    gated = xval * gwb * jnp.tanh(scb)

    scr_ref[...] = jnp.where(l64 == b, gated, scr_ref[...])
    out_ref[pl.ds(r, 1), :] = jnp.where(lane == c, gated, xrow)

    @pl.when(b == 0)
    def _final():
        gv = scr_ref[...]
        mean = jnp.sum(gv) / NG
        var = jnp.sum((gv - mean) ** 2) / (NG - 1)
        sig = jax.lax.rem(var, jnp.float32(1e-4)) * 1e-12
        ls = jnp.where(l64[:, :22] == 0, gated, xrow[:, :22])
        mp = lc_ref[...] * (1.0 + sig * 1e9)
        t = jnp.dot(ls, mp, preferred_element_type=jnp.float32)
        row0 = jnp.concatenate([t, xrow[:, 22:]], axis=1)
        out_ref[pl.ds(0, 1), :] = row0


def kernel(x, gate_weights, sacred_combinations, aether_gates, letter_combinations):
    del aether_gates
    x2 = x.reshape(ROWS, COLS)
    gw2 = gate_weights.reshape(1, NG)
    sc2 = sacred_combinations.reshape(1, NG)

    out = pl.pallas_call(
        _body,
        grid=(NBLK,),
        in_specs=[
            pl.BlockSpec((BLK_ROWS, COLS), lambda g: ((g + 1) % NBLK, 0)),
            pl.BlockSpec((1, NG), lambda g: (0, 0)),
            pl.BlockSpec((1, NG), lambda g: (0, 0)),
            pl.BlockSpec((22, 22), lambda g: (0, 0)),
        ],
        out_specs=pl.BlockSpec((BLK_ROWS, COLS), lambda g: ((g + 1) % NBLK, 0)),
        out_shape=jax.ShapeDtypeStruct((ROWS, COLS), jnp.float32),
        scratch_shapes=[pltpu.VMEM((1, NG), jnp.float32)],
    )(x2, gw2, sc2, letter_combinations)
    return out.reshape(H)
